# baseline (speedup 1.0000x reference)
import functools

import jax
import jax.numpy as jnp
from jax import lax
from jax.experimental import pallas as pl
from jax.experimental.pallas import tpu as pltpu
from jax.experimental.pallas import tpu_sc as plsc

_EPS = 0.0001
_DELT = (4 - 0) / 10
_NC, _NS, _L = 2, 16, 16
_NW = _NC * _NS
_TBL = 256

_N = 16384 * 200
_CHUNK = _N // _NW
_UNROLL = 16


def _step(table_ref, v):
    av = jnp.abs(v)
    ta = av.astype(jnp.int32)
    tfa = ta.astype(jnp.float32)
    ca = ta + jnp.where(av > tfa, 1, 0)
    neg = v < 0.0
    fidx = jnp.where(neg, ca, ta)
    cidx = jnp.where(neg, ta, ca)
    v2 = v + _EPS
    t2 = v2.astype(jnp.int32)
    tf2 = t2.astype(jnp.float32)
    fl2 = tf2 - jnp.where(v2 < tf2, 1.0, 0.0)
    ce2 = tf2 + jnp.where(v2 > tf2, 1.0, 0.0)
    a = plsc.load_gather(table_ref, [fidx])
    b = plsc.load_gather(table_ref, [cidx])
    ipol = a * (ce2 - v2 + _EPS) + b * (v2 - fl2 - _EPS)
    return v - (10.0 - ipol) * _DELT


def _body(v_hbm, dragf_hbm, out_hbm, table_v, vbuf, sem):
    wid = lax.axis_index("s") * _NC + lax.axis_index("c")
    base = wid * _CHUNK
    pltpu.sync_copy(dragf_hbm, table_v)
    pltpu.async_copy(v_hbm.at[pl.ds(base, _CHUNK)], vbuf, sem).wait()

    @plsc.parallel_loop(0, _CHUNK // _L, 1, unroll=_UNROLL)
    def loop_body(i):
        off = i * _L
        vv = vbuf[pl.ds(off, _L)]
        for _ in range(10):
            vv = _step(table_v, vv)
        vbuf[pl.ds(off, _L)] = vv
    pltpu.async_copy(vbuf, out_hbm.at[pl.ds(base, _CHUNK)], sem).wait()


@jax.jit
def _sc_run(vflat, dragf_pad):
    mesh = plsc.VectorSubcoreMesh(core_axis_name="c", subcore_axis_name="s",
                                  num_cores=_NC, num_subcores=_NS)
    return pl.kernel(
        _body,
        out_type=jax.ShapeDtypeStruct((_N,), jnp.float32),
        mesh=mesh,
        compiler_params=pltpu.CompilerParams(needs_layout_passes=False,
                                             disable_bounds_checks=True),
        scratch_types=[
            pltpu.VMEM((_TBL,), jnp.float32),
            pltpu.VMEM((_CHUNK,), jnp.float32),
            pltpu.SemaphoreType.DMA,
        ],
    )(vflat, dragf_pad)


def kernel(v, dragf):
    vflat = v.reshape(-1)
    dragf_pad = jnp.pad(dragf, (0, _TBL - dragf.shape[0]))
    return _sc_run(vflat, dragf_pad).reshape(v.shape)


# speedup vs baseline: 1.8214x; 1.8214x over previous
import functools

import jax
import jax.numpy as jnp
from jax import lax
from jax.experimental import pallas as pl
from jax.experimental.pallas import tpu as pltpu
from jax.experimental.pallas import tpu_sc as plsc

_EPS = 0.0001
_DELT = (4 - 0) / 10
_NC, _NS, _L = 2, 16, 16
_NW = _NC * _NS
_TBL = 256

_N = 16384 * 200
_CHUNK = _N // _NW
_UNROLL = 12


def _step(table_ref, v):
    av = jnp.abs(v)
    ta = av.astype(jnp.int32)
    tfa = ta.astype(jnp.float32)
    ca = ta + jnp.where(av > tfa, 1, 0)
    neg = v < 0.0
    fidx = jnp.where(neg, ca, ta)
    cidx = jnp.where(neg, ta, ca)
    v2 = v + _EPS
    t2 = v2.astype(jnp.int32)
    tf2 = t2.astype(jnp.float32)
    fl2 = tf2 - jnp.where(v2 < tf2, 1.0, 0.0)
    ce2 = tf2 + jnp.where(v2 > tf2, 1.0, 0.0)
    a = plsc.load_gather(table_ref, [fidx])
    b = plsc.load_gather(table_ref, [cidx])
    ipol = a * (ce2 - v2 + _EPS) + b * (v2 - fl2 - _EPS)
    return v - (10.0 - ipol) * _DELT


def _body(v_hbm, dragf_hbm, out_hbm, table_v, vbuf, sem):
    wid = lax.axis_index("s") * _NC + lax.axis_index("c")
    base = wid * _CHUNK
    pltpu.sync_copy(dragf_hbm, table_v)
    pltpu.async_copy(v_hbm.at[pl.ds(base, _CHUNK)], vbuf, sem).wait()

    @plsc.parallel_loop(0, _CHUNK // _L, 1, unroll=_UNROLL)
    def loop_body(i):
        off = i * _L
        vv = vbuf[pl.ds(off, _L)]
        for _ in range(10):
            vv = _step(table_v, vv)
        vbuf[pl.ds(off, _L)] = vv
    pltpu.async_copy(vbuf, out_hbm.at[pl.ds(base, _CHUNK)], sem).wait()


@jax.jit
def _sc_run(vflat, dragf_pad):
    mesh = plsc.VectorSubcoreMesh(core_axis_name="c", subcore_axis_name="s",
                                  num_cores=_NC, num_subcores=_NS)
    return pl.kernel(
        _body,
        out_type=jax.ShapeDtypeStruct((_N,), jnp.float32),
        mesh=mesh,
        compiler_params=pltpu.CompilerParams(needs_layout_passes=False,
                                             disable_bounds_checks=True),
        scratch_types=[
            pltpu.VMEM((_TBL,), jnp.float32),
            pltpu.VMEM((_CHUNK,), jnp.float32),
            pltpu.SemaphoreType.DMA,
        ],
    )(vflat, dragf_pad)


def kernel(v, dragf):
    vflat = v.reshape(-1)
    dragf_pad = jnp.pad(dragf, (0, _TBL - dragf.shape[0]))
    return _sc_run(vflat, dragf_pad).reshape(v.shape)
